# baseline (speedup 1.0000x reference)
import jax
import jax.numpy as jnp
from jax import lax
from jax.experimental import pallas as pl
from jax.experimental.pallas import tpu as pltpu
from jax.experimental.pallas import tpu_sc as plsc

_L = 200
_B = 1024
_M = 64
_K = 1000000
_KW = _K // 2
_SUB = 128
_UB = 256
_NQ = _B // _UB
_NR = _B // _SUB
_NU = 3 * (_L - 1) * _NQ
_NW = 32
_TC = 256
_NT = _K // _TC
_TTAIL = _K - _NT * _TC


def _wid():
    return lax.axis_index("s") * 2 + lax.axis_index("c")


def _tbody(embT_h, tail_h, wide_h, src0, src1, dst0, dst1, sem_l, sem_s):
    w = _wid()
    iota = lax.iota(jnp.int32, 16)
    srcs = (src0, src1)
    dsts = (dst0, dst1)
    mvecs = []
    pars = []
    for jc in range(8):
        mvecs.append((jc % 4) * 16 + iota)
        pars.append(1 if jc >= 4 else 0)

    nu = (_NT // _NW) + jnp.where(w < (_NT % _NW), 1, 0)

    def _r0(i):
        return pl.multiple_of((w + i * _NW) * _TC, _SUB)

    def _fire_load(i, b):
        pltpu.async_copy(embT_h.at[:, pl.ds(_r0(i), _TC)], srcs[b], sem_l)

    def _wait_load(b):
        pltpu.make_async_copy(embT_h.at[:, pl.ds(0, _TC)], srcs[b],
                              sem_l).wait()

    def _transpose(b):
        srcb = srcs[b]
        dstb = dsts[b]

        def _row(kk, carry):
            c2 = 2 * kk
            for jc in range(8):
                cvec = jnp.full((16,), c2 + pars[jc], jnp.int32)
                val = plsc.load_gather(srcb, [mvecs[jc], cvec])
                dstb[kk, pl.ds(jc * 16, 16)] = val
            return carry

        lax.fori_loop(0, _TC // 2, _row, 0)

    def _fire_store(i, b):
        o = pl.multiple_of((w + i * _NW) * (_TC // 2), _TC // 2)
        pltpu.async_copy(dsts[b], wide_h.at[pl.ds(o, _TC // 2)], sem_s)

    def _wait_store():
        pltpu.make_async_copy(wide_h.at[pl.ds(0, _TC // 2)], dst0,
                              sem_s).wait()

    _fire_load(0, 0)

    def _step(p, carry):
        i0 = 2 * p
        i1 = i0 + 1

        @pl.when(i1 < nu)
        def _():
            _fire_load(i1, 1)

        _wait_load(0)

        @pl.when(i0 >= 2)
        def _():
            _wait_store()

        _transpose(0)
        _fire_store(i0, 0)

        @pl.when(i1 < nu)
        def _():
            @pl.when(i1 + 1 < nu)
            def _():
                _fire_load(i1 + 1, 0)

            _wait_load(1)

            @pl.when(i1 >= 2)
            def _():
                _wait_store()

            _transpose(1)
            _fire_store(i1, 1)

        return carry

    lax.fori_loop(0, (nu + 1) // 2, _step, 0)
    _wait_store()
    _wait_store()

    @pl.when(w == 31)
    def _():
        pltpu.sync_copy(tail_h, dst0.at[pl.ds(0, _TTAIL // 2)])
        pltpu.sync_copy(dst0.at[pl.ds(0, _TTAIL // 2)],
                        wide_h.at[pl.ds(_NT * (_TC // 2), _TTAIL // 2)])


def _gbody(ly_h, lp_h, ry_h, lyg_h, lpg_h, ryg_h, wide_h, pos_h,
           lo_h, po_h, ro_h,
           ridx0, ridx1, gidx0, gidx1, wv0, wv1, ov0, ov1, sem_g, sem_s):
    w = _wid()
    iota = lax.iota(jnp.int32, 16)
    zvec = jnp.zeros((16,), jnp.float32)
    ridxs = (ridx0, ridx1)
    gidxs = (gidx0, gidx1)
    wvs = (wv0, wv1)
    ovs = (ov0, ov1)

    def _zrow(m, carry):
        for cc in range(_SUB // 16):
            ov0[m, pl.ds(cc * 16, 16)] = zvec
        return carry

    lax.fori_loop(0, _M, _zrow, 0)
    zq = pl.multiple_of((w % 8) * _SUB, _SUB)

    @pl.when(w < 8)
    def _():
        pltpu.sync_copy(ov0.at[:, pl.ds(0, _SUB)],
                        lo_h.at[0, :, pl.ds(zq, _SUB)])

    @pl.when(jnp.logical_and(w >= 8, w < 16))
    def _():
        pltpu.sync_copy(ov0.at[:, pl.ds(0, _SUB)],
                        po_h.at[0, :, pl.ds(zq, _SUB)])

    @pl.when(jnp.logical_and(w >= 16, w < 24))
    def _():
        pltpu.sync_copy(ov0.at[:, pl.ds(0, _SUB)],
                        ro_h.at[0, :, pl.ds(zq, _SUB)])

    nu = (_NU // _NW) + jnp.where(w < (_NU % _NW), 1, 0)

    def _split(uid):
        task = uid % 3
        rem = uid // 3
        l = 1 + rem // _NQ
        q = rem % _NQ
        return task, l, q

    def _fire(uid, b):
        task, l, q = _split(uid)
        row_s = pl.multiple_of((l - 1) * _NR, _NR)
        row_r = pl.multiple_of(l * _NR, _NR)

        def _one(r_h, g_h, tab_h, row):
            pltpu.sync_copy(r_h.at[pl.ds(row, _NR)], ridxs[b])
            pltpu.sync_copy(g_h.at[pl.ds(row, _NR)], gidxs[b])
            for j in range(_UB // _SUB):
                pltpu.async_copy(tab_h.at[gidxs[b].at[q * 2 + j]],
                                 wvs[b].at[pl.ds(j * _SUB, _SUB)], sem_g)

        @pl.when(task == 0)
        def _():
            _one(ly_h, lyg_h, wide_h, row_s)

        @pl.when(task == 1)
        def _():
            _one(lp_h, lpg_h, pos_h, row_s)

        @pl.when(task == 2)
        def _():
            _one(ry_h, ryg_h, wide_h, row_r)

    def _wait_g(uid, b):
        _, _, q = _split(uid)
        for j in range(_UB // _SUB):
            pltpu.make_async_copy(wide_h.at[gidxs[b].at[q * 2 + j]],
                                  wvs[b].at[pl.ds(j * _SUB, _SUB)],
                                  sem_g).wait()

    def _extract(uid, b):
        _, _, q = _split(uid)
        wvb = wvs[b]
        ovb = ovs[b]
        ridxb = ridxs[b]

        def _grp(g, carry):
            rvec = ridxb[q * 2 + g // 8, pl.ds((g % 8) * 16, 16)]
            rowvec = g * 16 + iota
            colbase = (rvec & 1) * 64
            for m in range(_M):
                val = plsc.load_gather(wvb, [rowvec, colbase + m])
                ovb[m, pl.ds(g * 16, 16)] = val
            return carry

        lax.fori_loop(0, _UB // 16, _grp, 0)

    def _store(uid, b):
        task, l, q = _split(uid)
        off = pl.multiple_of(q * _UB, _UB)

        @pl.when(task == 0)
        def _():
            pltpu.async_copy(ovs[b], lo_h.at[l, :, pl.ds(off, _UB)], sem_s)

        @pl.when(task == 1)
        def _():
            pltpu.async_copy(ovs[b], po_h.at[l, :, pl.ds(off, _UB)], sem_s)

        @pl.when(task == 2)
        def _():
            pltpu.async_copy(ovs[b], ro_h.at[l, :, pl.ds(off, _UB)], sem_s)

    def _wait_s():
        pltpu.make_async_copy(lo_h.at[0, :, pl.ds(0, _UB)], ov0,
                              sem_s).wait()

    _fire(w, 0)

    def _step(p, carry):
        i0 = 2 * p
        i1 = i0 + 1
        u0 = w + i0 * _NW
        u1 = w + i1 * _NW

        @pl.when(i1 < nu)
        def _():
            _fire(u1, 1)

        _wait_g(u0, 0)

        @pl.when(i0 >= 2)
        def _():
            _wait_s()

        _extract(u0, 0)
        _store(u0, 0)

        @pl.when(i1 < nu)
        def _():
            @pl.when(i1 + 1 < nu)
            def _():
                _fire(u1 + _NW, 0)

            _wait_g(u1, 1)

            @pl.when(i1 >= 2)
            def _():
                _wait_s()

            _extract(u1, 1)
            _store(u1, 1)

        return carry

    lax.fori_loop(0, (nu + 1) // 2, _step, 0)
    _wait_s()
    _wait_s()


@jax.jit
def kernel(ly, lp, ry, emb_table, pos_table):
    nr = _L * _B // _SUB
    ly2 = ly.astype(jnp.int32).reshape(nr, _SUB)
    lp2 = lp.astype(jnp.int32).reshape(nr, _SUB)
    ry2 = ry.astype(jnp.int32).reshape(nr, _SUB)
    lyg = (ly2 >> 1)
    lpg = (lp2 >> 1)
    ryg = (ry2 >> 1)
    embT = emb_table.T
    tail_w = emb_table[_NT * _TC:].reshape(_TTAIL // 2, 2 * _M)
    pos_w = pos_table.reshape(pos_table.shape[0] // 2, 2 * _M)

    mesh = plsc.VectorSubcoreMesh(core_axis_name="c", subcore_axis_name="s")

    trans = pl.kernel(
        _tbody,
        mesh=mesh,
        out_type=jax.ShapeDtypeStruct((_KW, 2 * _M), jnp.float32),
        scratch_types=[
            pltpu.VMEM((_M, _TC), jnp.float32),
            pltpu.VMEM((_M, _TC), jnp.float32),
            pltpu.VMEM((_TC // 2, 2 * _M), jnp.float32),
            pltpu.VMEM((_TC // 2, 2 * _M), jnp.float32),
            pltpu.SemaphoreType.DMA,
            pltpu.SemaphoreType.DMA,
        ],
        compiler_params=pltpu.CompilerParams(needs_layout_passes=False),
    )
    wide = trans(embT, tail_w)

    gather = pl.kernel(
        _gbody,
        mesh=mesh,
        out_type=(jax.ShapeDtypeStruct((_L, _M, _B), jnp.float32),) * 3,
        scratch_types=[
            pltpu.VMEM((_NR, _SUB), jnp.int32),
            pltpu.VMEM((_NR, _SUB), jnp.int32),
            pltpu.VMEM((_NR, _SUB), jnp.int32),
            pltpu.VMEM((_NR, _SUB), jnp.int32),
            pltpu.VMEM((_UB, _SUB), jnp.float32),
            pltpu.VMEM((_UB, _SUB), jnp.float32),
            pltpu.VMEM((_M, _UB), jnp.float32),
            pltpu.VMEM((_M, _UB), jnp.float32),
            pltpu.SemaphoreType.DMA,
            pltpu.SemaphoreType.DMA,
        ],
        compiler_params=pltpu.CompilerParams(needs_layout_passes=False),
    )
    lo, po, ro = gather(ly2, lp2, ry2, lyg, lpg, ryg, wide, pos_w)
    return (jnp.transpose(lo, (0, 2, 1)),
            jnp.transpose(po, (0, 2, 1)),
            jnp.transpose(ro, (0, 2, 1)))


# speedup vs baseline: 1.6910x; 1.6910x over previous
import jax
import jax.numpy as jnp
from jax import lax
from jax.experimental import pallas as pl
from jax.experimental.pallas import tpu as pltpu
from jax.experimental.pallas import tpu_sc as plsc

_L = 200
_B = 1024
_M = 64
_K = 1000000
_KW = _K // 2
_SUB = 128
_UB = 256
_NQ = _B // _UB
_NR = _B // _SUB
_NU = 3 * (_L - 1) * _NQ
_NW = 32
_TC = 256
_NT = _K // _TC
_TTAIL = _K - _NT * _TC


def _wid():
    return lax.axis_index("s") * 2 + lax.axis_index("c")


def _tbody(embT_h, tail_h, wide_h, src0, src1, dst0, dst1, sem_l, sem_s):
    w = _wid()
    iota = lax.iota(jnp.int32, 16)
    srcs = (src0, src1)
    dsts = (dst0, dst1)
    mvecs = []
    pars = []
    for jc in range(8):
        mvecs.append((jc % 4) * 16 + iota)
        pars.append(1 if jc >= 4 else 0)

    nu = (_NT // _NW) + jnp.where(w < (_NT % _NW), 1, 0)

    def _r0(i):
        return pl.multiple_of((w + i * _NW) * _TC, _SUB)

    def _fire_load(i, b):
        pltpu.async_copy(embT_h.at[:, pl.ds(_r0(i), _TC)], srcs[b], sem_l)

    def _wait_load(b):
        pltpu.make_async_copy(embT_h.at[:, pl.ds(0, _TC)], srcs[b],
                              sem_l).wait()

    def _transpose(b):
        srcb = srcs[b]
        dstb = dsts[b]

        @plsc.parallel_loop(0, _TC // 2, unroll=8)
        def _row(kk):
            c2 = 2 * kk
            for jc in range(8):
                cvec = jnp.full((16,), c2 + pars[jc], jnp.int32)
                val = plsc.load_gather(srcb, [mvecs[jc], cvec])
                dstb[kk, pl.ds(jc * 16, 16)] = val

    def _fire_store(i, b):
        o = pl.multiple_of((w + i * _NW) * (_TC // 2), _TC // 2)
        pltpu.async_copy(dsts[b], wide_h.at[pl.ds(o, _TC // 2)], sem_s)

    def _wait_store():
        pltpu.make_async_copy(wide_h.at[pl.ds(0, _TC // 2)], dst0,
                              sem_s).wait()

    _fire_load(0, 0)

    def _step(p, carry):
        i0 = 2 * p
        i1 = i0 + 1

        @pl.when(i1 < nu)
        def _():
            _fire_load(i1, 1)

        _wait_load(0)

        @pl.when(i0 >= 2)
        def _():
            _wait_store()

        _transpose(0)
        _fire_store(i0, 0)

        @pl.when(i1 < nu)
        def _():
            @pl.when(i1 + 1 < nu)
            def _():
                _fire_load(i1 + 1, 0)

            _wait_load(1)

            @pl.when(i1 >= 2)
            def _():
                _wait_store()

            _transpose(1)
            _fire_store(i1, 1)

        return carry

    lax.fori_loop(0, (nu + 1) // 2, _step, 0)
    _wait_store()
    _wait_store()

    @pl.when(w == 31)
    def _():
        pltpu.sync_copy(tail_h, dst0.at[pl.ds(0, _TTAIL // 2)])
        pltpu.sync_copy(dst0.at[pl.ds(0, _TTAIL // 2)],
                        wide_h.at[pl.ds(_NT * (_TC // 2), _TTAIL // 2)])


def _gbody(ly_h, lp_h, ry_h, lyg_h, lpg_h, ryg_h, wide_h, pos_h,
           lo_h, po_h, ro_h,
           ridx0, ridx1, gidx0, gidx1, wv0, wv1, ov0, ov1, sem_g, sem_s):
    w = _wid()
    iota = lax.iota(jnp.int32, 16)
    zvec = jnp.zeros((16,), jnp.float32)
    ridxs = (ridx0, ridx1)
    gidxs = (gidx0, gidx1)
    wvs = (wv0, wv1)
    ovs = (ov0, ov1)

    def _zrow(m, carry):
        for cc in range(_SUB // 16):
            ov0[m, pl.ds(cc * 16, 16)] = zvec
        return carry

    lax.fori_loop(0, _M, _zrow, 0)
    zq = pl.multiple_of((w % 8) * _SUB, _SUB)

    @pl.when(w < 8)
    def _():
        pltpu.sync_copy(ov0.at[:, pl.ds(0, _SUB)],
                        lo_h.at[0, :, pl.ds(zq, _SUB)])

    @pl.when(jnp.logical_and(w >= 8, w < 16))
    def _():
        pltpu.sync_copy(ov0.at[:, pl.ds(0, _SUB)],
                        po_h.at[0, :, pl.ds(zq, _SUB)])

    @pl.when(jnp.logical_and(w >= 16, w < 24))
    def _():
        pltpu.sync_copy(ov0.at[:, pl.ds(0, _SUB)],
                        ro_h.at[0, :, pl.ds(zq, _SUB)])

    nu = (_NU // _NW) + jnp.where(w < (_NU % _NW), 1, 0)

    def _split(uid):
        task = uid % 3
        rem = uid // 3
        l = 1 + rem // _NQ
        q = rem % _NQ
        return task, l, q

    def _fire(uid, b):
        task, l, q = _split(uid)
        row_s = pl.multiple_of((l - 1) * _NR, _NR)
        row_r = pl.multiple_of(l * _NR, _NR)

        def _one(r_h, g_h, tab_h, row):
            pltpu.sync_copy(r_h.at[pl.ds(row, _NR)], ridxs[b])
            pltpu.sync_copy(g_h.at[pl.ds(row, _NR)], gidxs[b])
            for j in range(_UB // _SUB):
                pltpu.async_copy(tab_h.at[gidxs[b].at[q * 2 + j]],
                                 wvs[b].at[pl.ds(j * _SUB, _SUB)], sem_g)

        @pl.when(task == 0)
        def _():
            _one(ly_h, lyg_h, wide_h, row_s)

        @pl.when(task == 1)
        def _():
            _one(lp_h, lpg_h, pos_h, row_s)

        @pl.when(task == 2)
        def _():
            _one(ry_h, ryg_h, wide_h, row_r)

    def _wait_g(uid, b):
        _, _, q = _split(uid)
        for j in range(_UB // _SUB):
            pltpu.make_async_copy(wide_h.at[gidxs[b].at[q * 2 + j]],
                                  wvs[b].at[pl.ds(j * _SUB, _SUB)],
                                  sem_g).wait()

    def _extract(uid, b):
        _, _, q = _split(uid)
        wvb = wvs[b]
        ovb = ovs[b]
        ridxb = ridxs[b]

        @plsc.parallel_loop(0, _UB // 16, unroll=2)
        def _grp(g):
            rvec = ridxb[q * 2 + g // 8, pl.ds((g % 8) * 16, 16)]
            rowvec = g * 16 + iota
            colbase = (rvec & 1) * 64
            for m in range(_M):
                val = plsc.load_gather(wvb, [rowvec, colbase + m])
                ovb[m, pl.ds(g * 16, 16)] = val

    def _store(uid, b):
        task, l, q = _split(uid)
        off = pl.multiple_of(q * _UB, _UB)

        @pl.when(task == 0)
        def _():
            pltpu.async_copy(ovs[b], lo_h.at[l, :, pl.ds(off, _UB)], sem_s)

        @pl.when(task == 1)
        def _():
            pltpu.async_copy(ovs[b], po_h.at[l, :, pl.ds(off, _UB)], sem_s)

        @pl.when(task == 2)
        def _():
            pltpu.async_copy(ovs[b], ro_h.at[l, :, pl.ds(off, _UB)], sem_s)

    def _wait_s():
        pltpu.make_async_copy(lo_h.at[0, :, pl.ds(0, _UB)], ov0,
                              sem_s).wait()

    _fire(w, 0)

    def _step(p, carry):
        i0 = 2 * p
        i1 = i0 + 1
        u0 = w + i0 * _NW
        u1 = w + i1 * _NW

        @pl.when(i1 < nu)
        def _():
            _fire(u1, 1)

        _wait_g(u0, 0)

        @pl.when(i0 >= 2)
        def _():
            _wait_s()

        _extract(u0, 0)
        _store(u0, 0)

        @pl.when(i1 < nu)
        def _():
            @pl.when(i1 + 1 < nu)
            def _():
                _fire(u1 + _NW, 0)

            _wait_g(u1, 1)

            @pl.when(i1 >= 2)
            def _():
                _wait_s()

            _extract(u1, 1)
            _store(u1, 1)

        return carry

    lax.fori_loop(0, (nu + 1) // 2, _step, 0)
    _wait_s()
    _wait_s()


@jax.jit
def kernel(ly, lp, ry, emb_table, pos_table):
    nr = _L * _B // _SUB
    ly2 = ly.astype(jnp.int32).reshape(nr, _SUB)
    lp2 = lp.astype(jnp.int32).reshape(nr, _SUB)
    ry2 = ry.astype(jnp.int32).reshape(nr, _SUB)
    lyg = (ly2 >> 1)
    lpg = (lp2 >> 1)
    ryg = (ry2 >> 1)
    embT = emb_table.T
    tail_w = emb_table[_NT * _TC:].reshape(_TTAIL // 2, 2 * _M)
    pos_w = pos_table.reshape(pos_table.shape[0] // 2, 2 * _M)

    mesh = plsc.VectorSubcoreMesh(core_axis_name="c", subcore_axis_name="s")

    trans = pl.kernel(
        _tbody,
        mesh=mesh,
        out_type=jax.ShapeDtypeStruct((_KW, 2 * _M), jnp.float32),
        scratch_types=[
            pltpu.VMEM((_M, _TC), jnp.float32),
            pltpu.VMEM((_M, _TC), jnp.float32),
            pltpu.VMEM((_TC // 2, 2 * _M), jnp.float32),
            pltpu.VMEM((_TC // 2, 2 * _M), jnp.float32),
            pltpu.SemaphoreType.DMA,
            pltpu.SemaphoreType.DMA,
        ],
        compiler_params=pltpu.CompilerParams(needs_layout_passes=False),
    )
    wide = trans(embT, tail_w)

    gather = pl.kernel(
        _gbody,
        mesh=mesh,
        out_type=(jax.ShapeDtypeStruct((_L, _M, _B), jnp.float32),) * 3,
        scratch_types=[
            pltpu.VMEM((_NR, _SUB), jnp.int32),
            pltpu.VMEM((_NR, _SUB), jnp.int32),
            pltpu.VMEM((_NR, _SUB), jnp.int32),
            pltpu.VMEM((_NR, _SUB), jnp.int32),
            pltpu.VMEM((_UB, _SUB), jnp.float32),
            pltpu.VMEM((_UB, _SUB), jnp.float32),
            pltpu.VMEM((_M, _UB), jnp.float32),
            pltpu.VMEM((_M, _UB), jnp.float32),
            pltpu.SemaphoreType.DMA,
            pltpu.SemaphoreType.DMA,
        ],
        compiler_params=pltpu.CompilerParams(needs_layout_passes=False),
    )
    lo, po, ro = gather(ly2, lp2, ry2, lyg, lpg, ryg, wide, pos_w)
    return (jnp.transpose(lo, (0, 2, 1)),
            jnp.transpose(po, (0, 2, 1)),
            jnp.transpose(ro, (0, 2, 1)))


# speedup vs baseline: 2.1625x; 1.2789x over previous
import jax
import jax.numpy as jnp
from jax import lax
from jax.experimental import pallas as pl
from jax.experimental.pallas import tpu as pltpu
from jax.experimental.pallas import tpu_sc as plsc

_L = 200
_B = 1024
_M = 64
_N = _L * _B
_SUB = 128
_UNIT = 1024
_NSUB = _UNIT // _SUB
_NTASK = 3
_TOT = _NTASK * (_L - 1)
_NW = 32
_ZROWS = _B // _NW


def _body(ly_h, lp_h, ry_h, emb_h, pos_h, lo_h, po_h, ro_h,
          idx_v, rows_v, sem):
    c = lax.axis_index("c")
    s = lax.axis_index("s")
    w = s * 2 + c

    zvec = jnp.zeros((16,), jnp.float32)

    def _zrow(r, carry):
        for cc in range(_M // 16):
            rows_v[r, pl.ds(cc * 16, 16)] = zvec
        return carry

    lax.fori_loop(0, _ZROWS, _zrow, 0)
    zbase = w * _ZROWS
    for out_h in (lo_h, po_h, ro_h):
        pltpu.sync_copy(rows_v.at[pl.ds(0, _ZROWS)],
                        out_h.at[0, pl.ds(zbase, _ZROWS), :])

    def _unit(idx_h, tab_h, out_h, irow, l):
        pltpu.sync_copy(idx_h.at[pl.ds(irow, _NSUB)], idx_v)
        descs = [
            pltpu.async_copy(tab_h.at[idx_v.at[j]],
                             rows_v.at[pl.ds(j * _SUB, _SUB)], sem)
            for j in range(_NSUB)
        ]
        for d in descs:
            d.wait()
        pltpu.sync_copy(rows_v, out_h.at[l])

    nu = (_TOT // _NW) + jnp.where(w < (_TOT % _NW), 1, 0)

    def _step(i, carry):
        uid = w + i * _NW
        task = uid % _NTASK
        l = 1 + uid // _NTASK
        irow = (l - 1) * _NSUB

        @pl.when(task == 0)
        def _():
            _unit(ly_h, emb_h, lo_h, irow, l)

        @pl.when(task == 1)
        def _():
            _unit(lp_h, pos_h, po_h, irow, l)

        @pl.when(task == 2)
        def _():
            _unit(ry_h, emb_h, ro_h, _NSUB + irow, l)

        return carry

    lax.fori_loop(0, nu, _step, 0)


@jax.jit
def kernel(ly, lp, ry, emb_table, pos_table):
    ly2 = ly.astype(jnp.int32).reshape(_N // _SUB, _SUB)
    lp2 = lp.astype(jnp.int32).reshape(_N // _SUB, _SUB)
    ry2 = ry.astype(jnp.int32).reshape(_N // _SUB, _SUB)

    mesh = plsc.VectorSubcoreMesh(core_axis_name="c", subcore_axis_name="s")
    out3 = (jax.ShapeDtypeStruct((_L, _B, _M), jnp.float32),) * 3
    run = pl.kernel(
        _body,
        mesh=mesh,
        out_type=out3,
        scratch_types=[
            pltpu.VMEM((_NSUB, _SUB), jnp.int32),
            pltpu.VMEM((_UNIT, _M), jnp.float32),
            pltpu.SemaphoreType.DMA,
        ],
        compiler_params=pltpu.CompilerParams(use_tc_tiling_on_sc=False),
    )
    return run(ly2, lp2, ry2, emb_table, pos_table)


# speedup vs baseline: 2.3984x; 1.1091x over previous
import jax
import jax.numpy as jnp
from jax import lax
from jax.experimental import pallas as pl
from jax.experimental.pallas import tpu as pltpu
from jax.experimental.pallas import tpu_sc as plsc

_L = 200
_B = 1024
_M = 64
_N = _L * _B
_SUB = 128
_UNIT = 1024
_NSUB = _UNIT // _SUB
_NW = 32
_ZROWS = _B // _NW


def _zero_fill(rows_v, outs, w):
    zvec = jnp.zeros((16,), jnp.float32)

    def _zrow(r, carry):
        for cc in range(_M // 16):
            rows_v[r, pl.ds(cc * 16, 16)] = zvec
        return carry

    lax.fori_loop(0, _ZROWS, _zrow, 0)
    zbase = w * _ZROWS
    for out_h in outs:
        pltpu.sync_copy(rows_v.at[pl.ds(0, _ZROWS)],
                        out_h.at[0, pl.ds(zbase, _ZROWS), :])


def _unit(idx_h, tab_h, out_h, idx_v, rows_v, sem, irow, l):
    pltpu.sync_copy(idx_h.at[pl.ds(irow, _NSUB)], idx_v)
    descs = [
        pltpu.async_copy(tab_h.at[idx_v.at[j]],
                         rows_v.at[pl.ds(j * _SUB, _SUB)], sem)
        for j in range(_NSUB)
    ]
    for d in descs:
        d.wait()
    pltpu.sync_copy(rows_v, out_h.at[l])


def _pbody(lp_h, pos_h, po_h, idx_v, rows_v, sem):
    w = lax.axis_index("s") * 2 + lax.axis_index("c")
    _zero_fill(rows_v, (po_h,), w)
    tot = _L - 1
    nu = (tot // _NW) + jnp.where(w < (tot % _NW), 1, 0)

    def _step(i, carry):
        l = 1 + w + i * _NW
        _unit(lp_h, pos_h, po_h, idx_v, rows_v, sem, (l - 1) * _NSUB, l)
        return carry

    lax.fori_loop(0, nu, _step, 0)


def _ebody(ly_h, ry_h, emb_h, lo_h, ro_h, idx_v, rows_v, sem):
    w = lax.axis_index("s") * 2 + lax.axis_index("c")
    _zero_fill(rows_v, (lo_h, ro_h), w)
    tot = 2 * (_L - 1)
    nu = (tot // _NW) + jnp.where(w < (tot % _NW), 1, 0)

    def _step(i, carry):
        uid = w + i * _NW
        task = uid % 2
        l = 1 + uid // 2
        irow = (l - 1) * _NSUB

        @pl.when(task == 0)
        def _():
            _unit(ly_h, emb_h, lo_h, idx_v, rows_v, sem, irow, l)

        @pl.when(task == 1)
        def _():
            _unit(ry_h, emb_h, ro_h, idx_v, rows_v, sem, _NSUB + irow, l)

        return carry

    lax.fori_loop(0, nu, _step, 0)


@jax.jit
def kernel(ly, lp, ry, emb_table, pos_table):
    ly2 = ly.astype(jnp.int32).reshape(_N // _SUB, _SUB)
    lp2 = lp.astype(jnp.int32).reshape(_N // _SUB, _SUB)
    ry2 = ry.astype(jnp.int32).reshape(_N // _SUB, _SUB)

    mesh = plsc.VectorSubcoreMesh(core_axis_name="c", subcore_axis_name="s")
    scratch = [
        pltpu.VMEM((_NSUB, _SUB), jnp.int32),
        pltpu.VMEM((_UNIT, _M), jnp.float32),
        pltpu.SemaphoreType.DMA,
    ]
    params = pltpu.CompilerParams(use_tc_tiling_on_sc=False)

    prun = pl.kernel(
        _pbody,
        mesh=mesh,
        out_type=jax.ShapeDtypeStruct((_L, _B, _M), jnp.float32),
        scratch_types=scratch,
        compiler_params=params,
    )
    po = prun(lp2, pos_table)

    erun = pl.kernel(
        _ebody,
        mesh=mesh,
        out_type=(jax.ShapeDtypeStruct((_L, _B, _M), jnp.float32),) * 2,
        scratch_types=scratch,
        compiler_params=params,
    )
    lo, ro = erun(ly2, ry2, emb_table)
    return (lo, po, ro)


# speedup vs baseline: 2.3987x; 1.0001x over previous
import jax
import jax.numpy as jnp
from jax import lax
from jax.experimental import pallas as pl
from jax.experimental.pallas import tpu as pltpu
from jax.experimental.pallas import tpu_sc as plsc

_L = 200
_B = 1024
_M = 64
_N = _L * _B
_SUB = 128
_UNIT = 1024
_NSUB = _UNIT // _SUB
_NW = 32
_ZROWS = _B // _NW


def _zero_fill(rows_v, outs, w):
    zvec = jnp.zeros((16,), jnp.float32)

    def _zrow(r, carry):
        for cc in range(_M // 16):
            rows_v[r, pl.ds(cc * 16, 16)] = zvec
        return carry

    lax.fori_loop(0, _ZROWS, _zrow, 0)
    zbase = w * _ZROWS
    for out_h in outs:
        pltpu.sync_copy(rows_v.at[pl.ds(0, _ZROWS)],
                        out_h.at[0, pl.ds(zbase, _ZROWS), :])


def _unit(idx_h, tab_h, out_h, idx_v, rows_v, sem, irow, l):
    pltpu.sync_copy(idx_h.at[pl.ds(irow, _NSUB)], idx_v)
    descs = [
        pltpu.async_copy(tab_h.at[idx_v.at[j]],
                         rows_v.at[pl.ds(j * _SUB, _SUB)], sem)
        for j in range(_NSUB)
    ]
    for d in descs:
        d.wait()
    pltpu.sync_copy(rows_v, out_h.at[l])


def _make_body(shifted):
    def _body(idx_h, tab_h, out_h, idx_v, rows_v, sem):
        w = lax.axis_index("s") * 2 + lax.axis_index("c")
        _zero_fill(rows_v, (out_h,), w)
        tot = _L - 1
        nu = (tot // _NW) + jnp.where(w < (tot % _NW), 1, 0)

        def _step(i, carry):
            l = 1 + w + i * _NW
            irow = (l - 1) * _NSUB if shifted else l * _NSUB
            _unit(idx_h, tab_h, out_h, idx_v, rows_v, sem, irow, l)
            return carry

        lax.fori_loop(0, nu, _step, 0)

    return _body


_sbody = _make_body(True)
_rbody = _make_body(False)


@jax.jit
def kernel(ly, lp, ry, emb_table, pos_table):
    ly2 = ly.astype(jnp.int32).reshape(_N // _SUB, _SUB)
    lp2 = lp.astype(jnp.int32).reshape(_N // _SUB, _SUB)
    ry2 = ry.astype(jnp.int32).reshape(_N // _SUB, _SUB)

    mesh = plsc.VectorSubcoreMesh(core_axis_name="c", subcore_axis_name="s")
    scratch = [
        pltpu.VMEM((_NSUB, _SUB), jnp.int32),
        pltpu.VMEM((_UNIT, _M), jnp.float32),
        pltpu.SemaphoreType.DMA,
    ]
    params = pltpu.CompilerParams(use_tc_tiling_on_sc=False)

    out1 = jax.ShapeDtypeStruct((_L, _B, _M), jnp.float32)

    def _call(body, name):
        return pl.kernel(
            body,
            mesh=mesh,
            out_type=out1,
            scratch_types=scratch,
            compiler_params=params,
            name=name,
        )

    po = _call(_sbody, "pos_gather")(lp2, pos_table)
    lo = _call(_sbody, "lemb_gather")(ly2, emb_table)
    ro = _call(_rbody, "remb_gather")(ry2, emb_table)
    return (lo, po, ro)
